# baseline (speedup 1.0000x reference)
import functools

import jax
import jax.numpy as jnp
from jax import lax
from jax.experimental import pallas as pl
from jax.experimental.pallas import tpu as pltpu
from jax.experimental.pallas import tpu_sc as plsc

_NC = 2
_NS = 16
_NW = _NC * _NS
_CH = 640


def _body(nch, d, idx_hbm, table_hbm, out_hbm, idx_v, rows0, rows1, sem0, sem1):
    wid = lax.axis_index("s") * _NC + lax.axis_index("c")
    pltpu.sync_copy(idx_hbm.at[wid], idx_v)
    base = wid * (nch * _CH)

    pltpu.async_copy(table_hbm.at[idx_v.at[0]], rows0, sem0)
    pltpu.async_copy(table_hbm.at[idx_v.at[1]], rows1, sem1)

    @pl.loop(0, nch - 2, step=2)
    def _(g):
        pltpu.make_async_copy(table_hbm.at[idx_v.at[g]], rows0, sem0).wait()
        pltpu.sync_copy(rows0, out_hbm.at[pl.ds(base + g * _CH, _CH)])
        pltpu.async_copy(table_hbm.at[idx_v.at[g + 2]], rows0, sem0)

        pltpu.make_async_copy(table_hbm.at[idx_v.at[g + 1]], rows1, sem1).wait()
        pltpu.sync_copy(rows1, out_hbm.at[pl.ds(base + (g + 1) * _CH, _CH)])
        pltpu.async_copy(table_hbm.at[idx_v.at[g + 3]], rows1, sem1)

    pltpu.make_async_copy(table_hbm.at[idx_v.at[nch - 2]], rows0, sem0).wait()
    pltpu.sync_copy(rows0, out_hbm.at[pl.ds(base + (nch - 2) * _CH, _CH)])
    pltpu.make_async_copy(table_hbm.at[idx_v.at[nch - 1]], rows1, sem1).wait()
    pltpu.sync_copy(rows1, out_hbm.at[pl.ds(base + (nch - 1) * _CH, _CH)])


@functools.partial(jax.jit, static_argnums=(2, 3, 4))
def _gather(idx, table, b, nch, d):
    mesh = plsc.VectorSubcoreMesh(core_axis_name="c", subcore_axis_name="s")
    k = pl.kernel(
        functools.partial(_body, nch, d),
        out_type=jax.ShapeDtypeStruct((b, d), jnp.float32),
        mesh=mesh,
        scratch_types=[
            pltpu.VMEM((nch, _CH), jnp.int32),
            pltpu.VMEM((_CH, d), jnp.float32),
            pltpu.VMEM((_CH, d), jnp.float32),
            pltpu.SemaphoreType.DMA,
            pltpu.SemaphoreType.DMA,
        ],
        compiler_params=pltpu.CompilerParams(use_tc_tiling_on_sc=False),
    )
    return k(idx, table)


def kernel(x, table):
    s, l = x.shape
    v, d = table.shape
    b = s * l
    nch = b // (_NW * _CH)
    idx = x.astype(jnp.int32).reshape(_NW, nch, _CH)
    out = _gather(idx, table, b, nch, d)
    return out.reshape(s, l, d)
